# Initial kernel scaffold; baseline (speedup 1.0000x reference)
import jax
import jax.numpy as jnp
from jax.experimental import pallas as pl


def kernel(node_features, edge_index, global_features, batch, W_in, b_in, gin_eps, gin_W1, gin_b1, gin_W2, gin_b2, W_comb, b_comb, Wd1, bd1, Wd2, bd2, Wd3, bd3, Wr1, br1, Wr2, br2, Wr3, br3):
    raise NotImplementedError("write your pallas kernel here")


# speedup vs baseline: 3.7091x; 3.7091x over previous
import functools

import jax
import jax.numpy as jnp
from jax import lax
from jax.experimental import pallas as pl
from jax.experimental.pallas import tpu as pltpu
from jax.experimental.pallas import tpu_sc as plsc

NC = 2
NS = 16
S = 128

_PREC = None


def _sc_agg(x, src, dst, *, n, half, hpad, zstride, ept):
  nb = ept // S
  nz = zstride // 32
  co = zstride
  co_last = half - (NS - 1) * co
  h = x.shape[1]

  mesh = plsc.VectorSubcoreMesh(
      core_axis_name="c", subcore_axis_name="s", num_cores=NC,
      num_subcores=NS)

  @functools.partial(
      pl.kernel,
      out_type=jax.ShapeDtypeStruct((n, h), jnp.float32),
      mesh=mesh,
      scratch_types=[
          pltpu.VMEM((S,), jnp.int32),
          pltpu.VMEM((S,), jnp.int32),
          pltpu.VMEM((S,), jnp.int32),
          pltpu.VMEM((S, h), jnp.float32),
          pltpu.VMEM((32, h), jnp.float32),
          pltpu.VMEM_SHARED((hpad, h), jnp.float32),
          pltpu.SemaphoreType.DMA,
      ],
      compiler_params=pltpu.CompilerParams(use_tc_tiling_on_sc=False),
  )
  def k(x_hbm, src_hbm, dst_hbm, out_hbm, srcb, dstb, mapb, rows, zbuf,
        aggsh, sem):
    cid = lax.axis_index("c")
    sid = lax.axis_index("s")
    base = cid * half

    for r in range(32):
      for c in range(h // 16):
        zbuf[r, pl.ds(c * 16, 16)] = jnp.zeros((16,), jnp.float32)

    def zloop(t, carry):
      pltpu.sync_copy(zbuf, aggsh.at[pl.ds(sid * zstride + t * 32, 32)])
      return carry
    lax.fori_loop(0, nz, zloop, 0)
    plsc.subcore_barrier()

    def chunk(i, carry):
      off = sid * ept + i * S
      pltpu.sync_copy(src_hbm.at[pl.ds(off, S)], srcb)
      pltpu.sync_copy(dst_hbm.at[pl.ds(off, S)], dstb)
      for j in range(S // 16):
        d = dstb[pl.ds(j * 16, 16)]
        ok = (d >= base) & (d < base + half)
        mapb[pl.ds(j * 16, 16)] = jnp.where(ok, d - base, half)
      pltpu.async_copy(x_hbm.at[srcb], rows, sem).wait()
      pltpu.sync_copy(rows, aggsh.at[mapb], add=True)
      return carry
    lax.fori_loop(0, nb, chunk, 0)
    plsc.subcore_barrier()

    @pl.when(sid < NS - 1)
    def _():
      pltpu.sync_copy(aggsh.at[pl.ds(sid * co, co)],
                      out_hbm.at[pl.ds(base + sid * co, co)])

    @pl.when(sid == NS - 1)
    def _():
      pltpu.sync_copy(aggsh.at[pl.ds((NS - 1) * co, co_last)],
                      out_hbm.at[pl.ds(base + (NS - 1) * co, co_last)])

  return k(x, src, dst)


def _tc_inproj(nf, w, b):
  n, d_in = nf.shape
  h = w.shape[1]
  r = 2000
  g = n // r

  def body(nf_ref, w_ref, b_ref, o_ref):
    o_ref[...] = jnp.maximum(
        jnp.dot(nf_ref[...], w_ref[...], preferred_element_type=jnp.float32,
                precision=_PREC) + b_ref[...], 0.0)

  return pl.pallas_call(
      body,
      grid=(g,),
      in_specs=[
          pl.BlockSpec((r, d_in), lambda i: (i, 0)),
          pl.BlockSpec((d_in, h), lambda i: (0, 0)),
          pl.BlockSpec((1, h), lambda i: (0, 0)),
      ],
      out_specs=pl.BlockSpec((r, h), lambda i: (i, 0)),
      out_shape=jax.ShapeDtypeStruct((n, h), jnp.float32),
  )(nf, w, b)


def _tc_layer(x, agg, eps, w1, b1, w2, b2, *, pool):
  n, h = x.shape
  r = 2000
  g = n // r

  def body(eps_ref, x_ref, a_ref, w1_ref, b1_ref, w2_ref, b2_ref, o_ref,
           *pool_refs):
    t = (1.0 + eps_ref[0]) * x_ref[...] + a_ref[...]
    t = jnp.maximum(
        jnp.dot(t, w1_ref[...], preferred_element_type=jnp.float32,
                precision=_PREC) + b1_ref[...], 0.0)
    t = jnp.maximum(
        jnp.dot(t, w2_ref[...], preferred_element_type=jnp.float32,
                precision=_PREC) + b2_ref[...], 0.0)
    o_ref[...] = t
    if pool_refs:
      p_ref = pool_refs[0]

      @pl.when(pl.program_id(0) == 0)
      def _():
        p_ref[...] = jnp.zeros_like(p_ref)

      p_ref[...] += jnp.sum(t, axis=0, keepdims=True)

  out_shape = [jax.ShapeDtypeStruct((n, h), jnp.float32)]
  out_specs = [pl.BlockSpec((r, h), lambda i: (i, 0))]
  if pool:
    out_shape.append(jax.ShapeDtypeStruct((1, h), jnp.float32))
    out_specs.append(pl.BlockSpec((1, h), lambda i: (0, 0)))

  return pl.pallas_call(
      body,
      grid=(g,),
      in_specs=[
          pl.BlockSpec(memory_space=pltpu.SMEM),
          pl.BlockSpec((r, h), lambda i: (i, 0)),
          pl.BlockSpec((r, h), lambda i: (i, 0)),
          pl.BlockSpec((h, h), lambda i: (0, 0)),
          pl.BlockSpec((1, h), lambda i: (0, 0)),
          pl.BlockSpec((h, h), lambda i: (0, 0)),
          pl.BlockSpec((1, h), lambda i: (0, 0)),
      ],
      out_specs=out_specs,
      out_shape=out_shape,
  )(eps, x, agg, w1, b1, w2, b2)


def _tc_heads(pooled_sum, n, gf, wc, bc, wd1, bd1, wd2, bd2, wd3, bd3,
              wr1, br1, wr2, br2, wr3, br3):
  h = pooled_sum.shape[1]
  d_g = gf.shape[1]
  nd = wd3.shape[1]
  nr = wr3.shape[1]

  def body(ps_ref, gf_ref, wc_ref, bc_ref, wd1_ref, bd1_ref, wd2_ref,
           bd2_ref, wd3_ref, bd3_ref, wr1_ref, br1_ref, wr2_ref, br2_ref,
           wr3_ref, br3_ref, od_ref, or_ref):
    def mm(a, w):
      return jnp.dot(a, w, preferred_element_type=jnp.float32,
                     precision=_PREC)

    pooled = ps_ref[...] * (1.0 / n)
    emb = jnp.maximum(
        mm(pooled, wc_ref[0:h, :]) + mm(gf_ref[...], wc_ref[h:h + d_g, :])
        + bc_ref[...], 0.0)
    d = jnp.maximum(mm(emb, wd1_ref[...]) + bd1_ref[...], 0.0)
    d = jnp.maximum(mm(d, wd2_ref[...]) + bd2_ref[...], 0.0)
    od_ref[...] = mm(d, wd3_ref[...]) + bd3_ref[...]
    rr = jnp.maximum(mm(emb, wr1_ref[...]) + br1_ref[...], 0.0)
    rr = jnp.maximum(mm(rr, wr2_ref[...]) + br2_ref[...], 0.0)
    or_ref[...] = mm(rr, wr3_ref[...]) + br3_ref[...]

  return pl.pallas_call(
      body,
      out_shape=(jax.ShapeDtypeStruct((1, nd), jnp.float32),
                 jax.ShapeDtypeStruct((1, nr), jnp.float32)),
  )(pooled_sum, gf, wc, bc, wd1, bd1, wd2, bd2, wd3, bd3,
    wr1, br1, wr2, br2, wr3, br3)


def kernel(node_features, edge_index, global_features, batch,
           W_in, b_in, gin_eps, gin_W1, gin_b1, gin_W2, gin_b2,
           W_comb, b_comb,
           Wd1, bd1, Wd2, bd2, Wd3, bd3,
           Wr1, br1, Wr2, br2, Wr3, br3):
  n, _ = node_features.shape
  e = edge_index.shape[1]
  h = W_in.shape[1]
  nl = gin_W1.shape[0]

  half = n // 2
  zstride = -(-(half + 1) // (NS * 32)) * 32
  hpad = NS * zstride
  ept = -(-e // (NS * S)) * S
  epad = NS * ept

  src = jnp.concatenate(
      [edge_index[0], jnp.zeros((epad - e,), jnp.int32)])
  dst = jnp.concatenate(
      [edge_index[1], jnp.full((epad - e,), n, jnp.int32)])

  x = _tc_inproj(node_features, W_in, b_in.reshape(1, h))
  pooled_sum = None
  for l in range(nl):
    agg = _sc_agg(x, src, dst, n=n, half=half, hpad=hpad,
                  zstride=zstride, ept=ept)
    res = _tc_layer(x, agg, gin_eps[l].reshape(1,),
                    gin_W1[l], gin_b1[l].reshape(1, h),
                    gin_W2[l], gin_b2[l].reshape(1, h),
                    pool=(l == nl - 1))
    if l == nl - 1:
      x, pooled_sum = res
    else:
      (x,) = res

  return _tc_heads(pooled_sum, n, global_features,
                   W_comb, b_comb.reshape(1, h),
                   Wd1, bd1.reshape(1, h), Wd2, bd2.reshape(1, h // 2),
                   Wd3, bd3.reshape(1, Wd3.shape[1]),
                   Wr1, br1.reshape(1, h), Wr2, br2.reshape(1, h // 2),
                   Wr3, br3.reshape(1, Wr3.shape[1]))


# speedup vs baseline: 7.8238x; 2.1093x over previous
import functools

import jax
import jax.numpy as jnp
from jax import lax
from jax.experimental import pallas as pl
from jax.experimental.pallas import tpu as pltpu
from jax.experimental.pallas import tpu_sc as plsc

NC = 2
NS = 16
S = 128

_PREC = None


def _sc_route(src, dst, *, n, half, ept0, rs):
  e_regions = 2 * NC * NS * rs
  nstage = ept0 // 1024

  mesh = plsc.VectorSubcoreMesh(
      core_axis_name="c", subcore_axis_name="s", num_cores=NC,
      num_subcores=NS)

  @functools.partial(
      pl.kernel,
      out_type=(jax.ShapeDtypeStruct((e_regions,), jnp.int32),
                jax.ShapeDtypeStruct((e_regions,), jnp.int32),
                jax.ShapeDtypeStruct((NC * NS * 16,), jnp.int32)),
      mesh=mesh,
      scratch_types=[
          pltpu.VMEM((1024,), jnp.int32),
          pltpu.VMEM((1024,), jnp.int32),
          pltpu.VMEM((rs + 16,), jnp.int32),
          pltpu.VMEM((rs + 16,), jnp.int32),
          pltpu.VMEM((rs + 16,), jnp.int32),
          pltpu.VMEM((rs + 16,), jnp.int32),
          pltpu.VMEM((16,), jnp.int32),
          pltpu.VMEM((16,), jnp.int32),
          pltpu.VMEM((16,), jnp.int32),
      ],
      compiler_params=pltpu.CompilerParams(needs_layout_passes=False),
  )
  def k(src_hbm, dst_hbm, rsrc_hbm, rdst_hbm, cnt_hbm,
        sstage, dstage, cs0, cd0, cs1, cd1, cnt, ob0, ob1):
    cid = lax.axis_index("c")
    sid = lax.axis_index("s")
    w = cid * NS + sid
    lane = lax.iota(jnp.int32, 16)

    zero16 = jnp.zeros((16,), jnp.int32)
    trash16 = jnp.full((16,), half, jnp.int32)

    def fill(i, carry):
      cs0[pl.ds(i * 16, 16)] = zero16
      cs1[pl.ds(i * 16, 16)] = zero16
      cd0[pl.ds(i * 16, 16)] = trash16
      cd1[pl.ds(i * 16, 16)] = trash16
      return carry
    lax.fori_loop(0, (rs + 16) // 16, fill, 0)
    ob0[pl.ds(0, 16)] = zero16
    ob1[pl.ds(0, 16)] = zero16

    def stage_loop(g, carry):
      eoff = w * ept0 + g * 1024
      pltpu.sync_copy(src_hbm.at[pl.ds(eoff, 1024)], sstage)
      pltpu.sync_copy(dst_hbm.at[pl.ds(eoff, 1024)], dstage)

      def vec_loop(kk, carry2):
        off0v = ob0[pl.ds(0, 16)]
        off1v = ob1[pl.ds(0, 16)]
        s16 = sstage[pl.ds(kk * 16, 16)]
        d16 = dstage[pl.ds(kk * 16, 16)]
        m0 = d16 < half
        m1 = (d16 >= half) & (d16 < n)
        i0 = jnp.where(m0, 1, 0)
        i1 = jnp.where(m1, 1, 0)
        r0v = plsc.cumsum(i0) - i0
        r1v = plsc.cumsum(i1) - i1
        garbage = jnp.full((16,), rs + 8, jnp.int32)
        idx0 = jnp.where(m0, off0v + r0v, garbage)
        idx1 = jnp.where(m1, off1v + r1v, garbage)
        plsc.store_scatter(cs0, [idx0], s16)
        plsc.store_scatter(cd0, [idx0], d16)
        plsc.store_scatter(cs1, [idx1], s16)
        plsc.store_scatter(cd1, [idx1], d16 - half)
        ob0[pl.ds(0, 16)] = off0v + plsc.all_reduce_population_count(m0)
        ob1[pl.ds(0, 16)] = off1v + plsc.all_reduce_population_count(m1)
        return carry2
      return lax.fori_loop(0, 64, vec_loop, carry)

    lax.fori_loop(0, nstage, stage_loop, 0)
    off0 = jnp.max(ob0[pl.ds(0, 16)])
    off1 = jnp.max(ob1[pl.ds(0, 16)])

    r0 = w * 2
    pltpu.sync_copy(cs0.at[pl.ds(0, rs)], rsrc_hbm.at[pl.ds(r0 * rs, rs)])
    pltpu.sync_copy(cd0.at[pl.ds(0, rs)], rdst_hbm.at[pl.ds(r0 * rs, rs)])
    pltpu.sync_copy(cs1.at[pl.ds(0, rs)],
                    rsrc_hbm.at[pl.ds((r0 + 1) * rs, rs)])
    pltpu.sync_copy(cd1.at[pl.ds(0, rs)],
                    rdst_hbm.at[pl.ds((r0 + 1) * rs, rs)])
    n0 = lax.shift_right_logical(off0 + (S - 1), 7)
    n1 = lax.shift_right_logical(off1 + (S - 1), 7)
    cvec = jnp.where(lane == 0, n0, jnp.where(lane == 1, n1, 0))
    cnt[pl.ds(0, 16)] = cvec
    pltpu.sync_copy(cnt, cnt_hbm.at[pl.ds(w * 16, 16)])

  return k(src, dst)


def _sc_agg_routed(x, rsrc2, rdst2, cnts, *, half, hpad, zstride, rs):
  co = zstride
  co_last = half - (NS - 1) * co
  h = x.shape[1]
  rrows = rs // S
  K = 2

  mesh = plsc.VectorSubcoreMesh(
      core_axis_name="c", subcore_axis_name="s", num_cores=NC,
      num_subcores=NS)

  @functools.partial(
      pl.kernel,
      out_type=jax.ShapeDtypeStruct((2 * half, h), jnp.float32),
      mesh=mesh,
      scratch_types=[
          pltpu.VMEM((K, S), jnp.int32),
          pltpu.VMEM((K, S), jnp.int32),
          pltpu.VMEM((S, h), jnp.float32),
          pltpu.VMEM((S, h), jnp.float32),
          pltpu.VMEM((16,), jnp.int32),
          pltpu.VMEM((16,), jnp.int32),
          pltpu.VMEM((32, h), jnp.float32),
          pltpu.VMEM_SHARED((hpad, h), jnp.float32),
          pltpu.SemaphoreType.DMA,
          pltpu.SemaphoreType.DMA,
      ],
      compiler_params=pltpu.CompilerParams(use_tc_tiling_on_sc=False,
                                           needs_layout_passes=False),
  )
  def k(x_hbm, rsrc_hbm, rdst_hbm, cnt_hbm, out_hbm,
        srcb, dstb, rows0, rows1, cb0, cb1, zbuf,
        aggsh, sg0, sg1):
    rows = (rows0, rows1)
    sg = (sg0, sg1)
    cid = lax.axis_index("c")
    sid = lax.axis_index("s")
    base = cid * half
    lane = lax.iota(jnp.int32, 16)

    for r in range(32):
      for c in range(h // 16):
        zbuf[r, pl.ds(c * 16, 16)] = jnp.zeros((16,), jnp.float32)

    def zloop(t, carry):
      pltpu.sync_copy(zbuf, aggsh.at[pl.ds(sid * zstride + t * 32, 32)])
      return carry
    lax.fori_loop(0, zstride // 32, zloop, 0)
    plsc.subcore_barrier()

    pltpu.sync_copy(cnt_hbm.at[pl.ds((2 * sid) * 16, 16)], cb0)
    pltpu.sync_copy(cnt_hbm.at[pl.ds((2 * sid + 1) * 16, 16)], cb1)
    trip0 = jnp.max(jnp.where(lane == cid, cb0[pl.ds(0, 16)], 0))
    trip1 = jnp.max(jnp.where(lane == cid, cb1[pl.ds(0, 16)], 0))
    r0 = 4 * sid + cid
    r1 = 4 * sid + 2 + cid

    for rid, trip in ((r0, trip0), (r1, trip1)):
      rowbase = rid * rrows

      def group(g, carry, rowbase=rowbase, trip=trip):
        pltpu.sync_copy(rsrc_hbm.at[pl.ds(rowbase + g * K, K)], srcb)
        pltpu.sync_copy(rdst_hbm.at[pl.ds(rowbase + g * K, K)], dstb)
        descs = [pltpu.make_async_copy(x_hbm.at[srcb.at[j]], rows[j],
                                       sg[j]) for j in range(K)]
        for j in range(K):
          @pl.when(g * K + j < trip)
          def _(j=j):
            descs[j].start()
        for j in range(K):
          @pl.when(g * K + j < trip)
          def _(j=j):
            descs[j].wait()
            pltpu.sync_copy(rows[j], aggsh.at[dstb.at[j]], add=True)
        return carry
      lax.fori_loop(0, lax.shift_right_logical(trip + 1, 1), group, 0)
    plsc.subcore_barrier()

    @pl.when(sid < NS - 1)
    def _():
      pltpu.sync_copy(aggsh.at[pl.ds(sid * co, co)],
                      out_hbm.at[pl.ds(base + sid * co, co)])

    @pl.when(sid == NS - 1)
    def _():
      pltpu.sync_copy(aggsh.at[pl.ds((NS - 1) * co, co_last)],
                      out_hbm.at[pl.ds(base + (NS - 1) * co, co_last)])

  return k(x, rsrc2, rdst2, cnts)


def _tc_inproj(nf, w, b):
  n, d_in = nf.shape
  h = w.shape[1]
  r = 2000
  g = n // r

  def body(nf_ref, w_ref, b_ref, o_ref):
    o_ref[...] = jnp.maximum(
        jnp.dot(nf_ref[...], w_ref[...], preferred_element_type=jnp.float32,
                precision=_PREC) + b_ref[...], 0.0)

  return pl.pallas_call(
      body,
      grid=(g,),
      in_specs=[
          pl.BlockSpec((r, d_in), lambda i: (i, 0)),
          pl.BlockSpec((d_in, h), lambda i: (0, 0)),
          pl.BlockSpec((1, h), lambda i: (0, 0)),
      ],
      out_specs=pl.BlockSpec((r, h), lambda i: (i, 0)),
      out_shape=jax.ShapeDtypeStruct((n, h), jnp.float32),
  )(nf, w, b)


def _tc_layer(x, agg, eps, w1, b1, w2, b2, *, pool):
  n, h = x.shape
  r = 2000
  g = n // r

  def body(eps_ref, x_ref, a_ref, w1_ref, b1_ref, w2_ref, b2_ref, o_ref,
           *pool_refs):
    t = (1.0 + eps_ref[0]) * x_ref[...] + a_ref[...]
    t = jnp.maximum(
        jnp.dot(t, w1_ref[...], preferred_element_type=jnp.float32,
                precision=_PREC) + b1_ref[...], 0.0)
    t = jnp.maximum(
        jnp.dot(t, w2_ref[...], preferred_element_type=jnp.float32,
                precision=_PREC) + b2_ref[...], 0.0)
    o_ref[...] = t
    if pool_refs:
      p_ref = pool_refs[0]

      @pl.when(pl.program_id(0) == 0)
      def _():
        p_ref[...] = jnp.zeros_like(p_ref)

      p_ref[...] += jnp.sum(t, axis=0, keepdims=True)

  out_shape = [jax.ShapeDtypeStruct((n, h), jnp.float32)]
  out_specs = [pl.BlockSpec((r, h), lambda i: (i, 0))]
  if pool:
    out_shape.append(jax.ShapeDtypeStruct((1, h), jnp.float32))
    out_specs.append(pl.BlockSpec((1, h), lambda i: (0, 0)))

  return pl.pallas_call(
      body,
      grid=(g,),
      in_specs=[
          pl.BlockSpec(memory_space=pltpu.SMEM),
          pl.BlockSpec((r, h), lambda i: (i, 0)),
          pl.BlockSpec((r, h), lambda i: (i, 0)),
          pl.BlockSpec((h, h), lambda i: (0, 0)),
          pl.BlockSpec((1, h), lambda i: (0, 0)),
          pl.BlockSpec((h, h), lambda i: (0, 0)),
          pl.BlockSpec((1, h), lambda i: (0, 0)),
      ],
      out_specs=out_specs,
      out_shape=out_shape,
  )(eps, x, agg, w1, b1, w2, b2)


def _tc_heads(pooled_sum, n, gf, wc, bc, wd1, bd1, wd2, bd2, wd3, bd3,
              wr1, br1, wr2, br2, wr3, br3):
  h = pooled_sum.shape[1]
  d_g = gf.shape[1]
  nd = wd3.shape[1]
  nr = wr3.shape[1]

  def body(ps_ref, gf_ref, wc_ref, bc_ref, wd1_ref, bd1_ref, wd2_ref,
           bd2_ref, wd3_ref, bd3_ref, wr1_ref, br1_ref, wr2_ref, br2_ref,
           wr3_ref, br3_ref, od_ref, or_ref):
    def mm(a, w):
      return jnp.dot(a, w, preferred_element_type=jnp.float32,
                     precision=_PREC)

    pooled = ps_ref[...] * (1.0 / n)
    emb = jnp.maximum(
        mm(pooled, wc_ref[0:h, :]) + mm(gf_ref[...], wc_ref[h:h + d_g, :])
        + bc_ref[...], 0.0)
    d = jnp.maximum(mm(emb, wd1_ref[...]) + bd1_ref[...], 0.0)
    d = jnp.maximum(mm(d, wd2_ref[...]) + bd2_ref[...], 0.0)
    od_ref[...] = mm(d, wd3_ref[...]) + bd3_ref[...]
    rr = jnp.maximum(mm(emb, wr1_ref[...]) + br1_ref[...], 0.0)
    rr = jnp.maximum(mm(rr, wr2_ref[...]) + br2_ref[...], 0.0)
    or_ref[...] = mm(rr, wr3_ref[...]) + br3_ref[...]

  return pl.pallas_call(
      body,
      out_shape=(jax.ShapeDtypeStruct((1, nd), jnp.float32),
                 jax.ShapeDtypeStruct((1, nr), jnp.float32)),
  )(pooled_sum, gf, wc, bc, wd1, bd1, wd2, bd2, wd3, bd3,
    wr1, br1, wr2, br2, wr3, br3)


def kernel(node_features, edge_index, global_features, batch,
           W_in, b_in, gin_eps, gin_W1, gin_b1, gin_W2, gin_b2,
           W_comb, b_comb,
           Wd1, bd1, Wd2, bd2, Wd3, bd3,
           Wr1, br1, Wr2, br2, Wr3, br3):
  n, _ = node_features.shape
  e = edge_index.shape[1]
  h = W_in.shape[1]
  nl = gin_W1.shape[0]

  half = n // 2
  zstride = -(-(half + 1) // (NS * 32)) * 32
  hpad = NS * zstride
  ept0 = -(-e // (NC * NS * 1024)) * 1024
  epad = NC * NS * ept0
  rs = ept0

  src = jnp.concatenate(
      [edge_index[0], jnp.zeros((epad - e,), jnp.int32)])
  dst = jnp.concatenate(
      [edge_index[1], jnp.full((epad - e,), n, jnp.int32)])

  rsrc, rdst, cnts = _sc_route(src, dst, n=n, half=half, ept0=ept0, rs=rs)
  rsrc = rsrc.reshape(-1, S)
  rdst = rdst.reshape(-1, S)

  x = _tc_inproj(node_features, W_in, b_in.reshape(1, h))
  pooled_sum = None
  for l in range(nl):
    agg = _sc_agg_routed(x, rsrc, rdst, cnts, half=half, hpad=hpad,
                         zstride=zstride, rs=rs)
    res = _tc_layer(x, agg, gin_eps[l].reshape(1,),
                    gin_W1[l], gin_b1[l].reshape(1, h),
                    gin_W2[l], gin_b2[l].reshape(1, h),
                    pool=(l == nl - 1))
    if l == nl - 1:
      x, pooled_sum = res
    else:
      (x,) = res

  return _tc_heads(pooled_sum, n, global_features,
                   W_comb, b_comb.reshape(1, h),
                   Wd1, bd1.reshape(1, h), Wd2, bd2.reshape(1, h // 2),
                   Wd3, bd3.reshape(1, Wd3.shape[1]),
                   Wr1, br1.reshape(1, h), Wr2, br2.reshape(1, h // 2),
                   Wr3, br3.reshape(1, Wr3.shape[1]))


# speedup vs baseline: 8.7033x; 1.1124x over previous
import functools

import jax
import jax.numpy as jnp
from jax import lax
from jax.experimental import pallas as pl
from jax.experimental.pallas import tpu as pltpu
from jax.experimental.pallas import tpu_sc as plsc

NC = 2
NS = 16
S = 128

_PREC = None


def _sc_route(src, dst, *, n, half, ept0, rs):
  e_regions = 2 * NC * NS * rs
  nstage = ept0 // 1024

  mesh = plsc.VectorSubcoreMesh(
      core_axis_name="c", subcore_axis_name="s", num_cores=NC,
      num_subcores=NS)

  @functools.partial(
      pl.kernel,
      out_type=(jax.ShapeDtypeStruct((e_regions,), jnp.int32),
                jax.ShapeDtypeStruct((e_regions,), jnp.int32),
                jax.ShapeDtypeStruct((NC * NS * 16,), jnp.int32)),
      mesh=mesh,
      scratch_types=[
          pltpu.VMEM((1024,), jnp.int32),
          pltpu.VMEM((1024,), jnp.int32),
          pltpu.VMEM((rs + 16,), jnp.int32),
          pltpu.VMEM((rs + 16,), jnp.int32),
          pltpu.VMEM((rs + 16,), jnp.int32),
          pltpu.VMEM((rs + 16,), jnp.int32),
          pltpu.VMEM((16,), jnp.int32),
          pltpu.VMEM((16,), jnp.int32),
          pltpu.VMEM((16,), jnp.int32),
      ],
      compiler_params=pltpu.CompilerParams(needs_layout_passes=False),
  )
  def k(src_hbm, dst_hbm, rsrc_hbm, rdst_hbm, cnt_hbm,
        sstage, dstage, cs0, cd0, cs1, cd1, cnt, ob0, ob1):
    cid = lax.axis_index("c")
    sid = lax.axis_index("s")
    w = cid * NS + sid
    lane = lax.iota(jnp.int32, 16)

    zero16 = jnp.zeros((16,), jnp.int32)
    trash16 = jnp.full((16,), half, jnp.int32)

    def fill(i, carry):
      cs0[pl.ds(i * 16, 16)] = zero16
      cs1[pl.ds(i * 16, 16)] = zero16
      cd0[pl.ds(i * 16, 16)] = trash16
      cd1[pl.ds(i * 16, 16)] = trash16
      return carry
    lax.fori_loop(0, (rs + 16) // 16, fill, 0)
    ob0[pl.ds(0, 16)] = zero16
    ob1[pl.ds(0, 16)] = zero16

    def stage_loop(g, carry):
      eoff = w * ept0 + g * 1024
      pltpu.sync_copy(src_hbm.at[pl.ds(eoff, 1024)], sstage)
      pltpu.sync_copy(dst_hbm.at[pl.ds(eoff, 1024)], dstage)

      def vec_loop(kk, carry2):
        off0v = ob0[pl.ds(0, 16)]
        off1v = ob1[pl.ds(0, 16)]
        s16 = sstage[pl.ds(kk * 16, 16)]
        d16 = dstage[pl.ds(kk * 16, 16)]
        m0 = d16 < half
        m1 = (d16 >= half) & (d16 < n)
        i0 = jnp.where(m0, 1, 0)
        i1 = jnp.where(m1, 1, 0)
        r0v = plsc.cumsum(i0) - i0
        r1v = plsc.cumsum(i1) - i1
        garbage = jnp.full((16,), rs + 8, jnp.int32)
        idx0 = jnp.where(m0, off0v + r0v, garbage)
        idx1 = jnp.where(m1, off1v + r1v, garbage)
        plsc.store_scatter(cs0, [idx0], s16)
        plsc.store_scatter(cd0, [idx0], d16)
        plsc.store_scatter(cs1, [idx1], s16)
        plsc.store_scatter(cd1, [idx1], d16 - half)
        ob0[pl.ds(0, 16)] = off0v + plsc.all_reduce_population_count(m0)
        ob1[pl.ds(0, 16)] = off1v + plsc.all_reduce_population_count(m1)
        return carry2
      return lax.fori_loop(0, 64, vec_loop, carry)

    lax.fori_loop(0, nstage, stage_loop, 0)
    off0 = jnp.max(ob0[pl.ds(0, 16)])
    off1 = jnp.max(ob1[pl.ds(0, 16)])

    r0 = w * 2
    pltpu.sync_copy(cs0.at[pl.ds(0, rs)], rsrc_hbm.at[pl.ds(r0 * rs, rs)])
    pltpu.sync_copy(cd0.at[pl.ds(0, rs)], rdst_hbm.at[pl.ds(r0 * rs, rs)])
    pltpu.sync_copy(cs1.at[pl.ds(0, rs)],
                    rsrc_hbm.at[pl.ds((r0 + 1) * rs, rs)])
    pltpu.sync_copy(cd1.at[pl.ds(0, rs)],
                    rdst_hbm.at[pl.ds((r0 + 1) * rs, rs)])
    n0 = lax.shift_right_logical(off0 + (S - 1), 7)
    n1 = lax.shift_right_logical(off1 + (S - 1), 7)
    cvec = jnp.where(lane == 0, n0, jnp.where(lane == 1, n1, 0))
    cnt[pl.ds(0, 16)] = cvec
    pltpu.sync_copy(cnt, cnt_hbm.at[pl.ds(w * 16, 16)])

  return k(src, dst)


def _sc_agg_routed(x, rsrc2, rdst2, cnts, *, half, hpad, zstride, rs):
  co = zstride
  co_last = half - (NS - 1) * co
  h = x.shape[1]
  rrows = rs // S
  K = 4

  mesh = plsc.VectorSubcoreMesh(
      core_axis_name="c", subcore_axis_name="s", num_cores=NC,
      num_subcores=NS)

  @functools.partial(
      pl.kernel,
      out_type=jax.ShapeDtypeStruct((2 * half, h), jnp.float32),
      mesh=mesh,
      scratch_types=[
          pltpu.VMEM((K, S), jnp.int32),
          pltpu.VMEM((K, S), jnp.int32),
          pltpu.VMEM((S, h), jnp.float32),
          pltpu.VMEM((S, h), jnp.float32),
          pltpu.VMEM((16,), jnp.int32),
          pltpu.VMEM((16,), jnp.int32),
          pltpu.VMEM((32, h), jnp.float32),
          pltpu.VMEM_SHARED((hpad, h), jnp.float32),
          pltpu.SemaphoreType.DMA,
          pltpu.SemaphoreType.DMA,
          pltpu.SemaphoreType.DMA,
          pltpu.SemaphoreType.DMA,
      ],
      compiler_params=pltpu.CompilerParams(use_tc_tiling_on_sc=False,
                                           needs_layout_passes=False),
  )
  def k(x_hbm, rsrc_hbm, rdst_hbm, cnt_hbm, out_hbm,
        srcb, dstb, rows0, rows1, cb0, cb1, zbuf,
        aggsh, sg0, sg1, ss0, ss1):
    rows = (rows0, rows1)
    sg = (sg0, sg1)
    ss = (ss0, ss1)
    cid = lax.axis_index("c")
    sid = lax.axis_index("s")
    base = cid * half
    lane = lax.iota(jnp.int32, 16)

    for r in range(32):
      for c in range(h // 16):
        zbuf[r, pl.ds(c * 16, 16)] = jnp.zeros((16,), jnp.float32)

    def zloop(t, carry):
      pltpu.sync_copy(zbuf, aggsh.at[pl.ds(sid * zstride + t * 32, 32)])
      return carry
    lax.fori_loop(0, zstride // 32, zloop, 0)
    plsc.subcore_barrier()

    pltpu.sync_copy(cnt_hbm.at[pl.ds((2 * sid) * 16, 16)], cb0)
    pltpu.sync_copy(cnt_hbm.at[pl.ds((2 * sid + 1) * 16, 16)], cb1)
    trip0 = jnp.max(jnp.where(lane == cid, cb0[pl.ds(0, 16)], 0))
    trip1 = jnp.max(jnp.where(lane == cid, cb1[pl.ds(0, 16)], 0))
    r0 = 4 * sid + cid
    r1 = 4 * sid + 2 + cid

    for rid, trip in ((r0, trip0), (r1, trip1)):
      rowbase = rid * rrows

      def group(g, carry, rowbase=rowbase, trip=trip):
        pltpu.sync_copy(rsrc_hbm.at[pl.ds(rowbase + g * K, K)], srcb)
        pltpu.sync_copy(rdst_hbm.at[pl.ds(rowbase + g * K, K)], dstb)
        full = (g * K + K) <= trip

        @pl.when(full)
        def _():
          d0 = pltpu.async_copy(x_hbm.at[srcb.at[0]], rows[0], sg[0])
          d1 = pltpu.async_copy(x_hbm.at[srcb.at[1]], rows[1], sg[1])
          d0.wait()
          s0 = pltpu.async_copy(rows[0], aggsh.at[dstb.at[0]], ss[0],
                                add=True)
          d1.wait()
          s1 = pltpu.async_copy(rows[1], aggsh.at[dstb.at[1]], ss[1],
                                add=True)
          s0.wait()
          d2 = pltpu.async_copy(x_hbm.at[srcb.at[2]], rows[0], sg[0])
          s1.wait()
          d3 = pltpu.async_copy(x_hbm.at[srcb.at[3]], rows[1], sg[1])
          d2.wait()
          s2 = pltpu.async_copy(rows[0], aggsh.at[dstb.at[2]], ss[0],
                                add=True)
          d3.wait()
          s3 = pltpu.async_copy(rows[1], aggsh.at[dstb.at[3]], ss[1],
                                add=True)
          s2.wait()
          s3.wait()

        @pl.when(jnp.logical_not(full))
        def _():
          for j in range(K):
            @pl.when(g * K + j < trip)
            def _(j=j):
              pltpu.async_copy(x_hbm.at[srcb.at[j]], rows[0],
                               sg[0]).wait()
              pltpu.sync_copy(rows[0], aggsh.at[dstb.at[j]], add=True)
        return carry
      lax.fori_loop(0, lax.shift_right_logical(trip + K - 1, 2), group, 0)
    plsc.subcore_barrier()

    @pl.when(sid < NS - 1)
    def _():
      pltpu.sync_copy(aggsh.at[pl.ds(sid * co, co)],
                      out_hbm.at[pl.ds(base + sid * co, co)])

    @pl.when(sid == NS - 1)
    def _():
      pltpu.sync_copy(aggsh.at[pl.ds((NS - 1) * co, co_last)],
                      out_hbm.at[pl.ds(base + (NS - 1) * co, co_last)])

  return k(x, rsrc2, rdst2, cnts)


def _tc_inproj(nf, w, b):
  n, d_in = nf.shape
  h = w.shape[1]
  r = 2000
  g = n // r

  def body(nf_ref, w_ref, b_ref, o_ref):
    o_ref[...] = jnp.maximum(
        jnp.dot(nf_ref[...], w_ref[...], preferred_element_type=jnp.float32,
                precision=_PREC) + b_ref[...], 0.0)

  return pl.pallas_call(
      body,
      grid=(g,),
      in_specs=[
          pl.BlockSpec((r, d_in), lambda i: (i, 0)),
          pl.BlockSpec((d_in, h), lambda i: (0, 0)),
          pl.BlockSpec((1, h), lambda i: (0, 0)),
      ],
      out_specs=pl.BlockSpec((r, h), lambda i: (i, 0)),
      out_shape=jax.ShapeDtypeStruct((n, h), jnp.float32),
  )(nf, w, b)


def _tc_layer(x, agg, eps, w1, b1, w2, b2, *, pool):
  n, h = x.shape
  r = 2000
  g = n // r

  def body(eps_ref, x_ref, a_ref, w1_ref, b1_ref, w2_ref, b2_ref, o_ref,
           *pool_refs):
    t = (1.0 + eps_ref[0]) * x_ref[...] + a_ref[...]
    t = jnp.maximum(
        jnp.dot(t, w1_ref[...], preferred_element_type=jnp.float32,
                precision=_PREC) + b1_ref[...], 0.0)
    t = jnp.maximum(
        jnp.dot(t, w2_ref[...], preferred_element_type=jnp.float32,
                precision=_PREC) + b2_ref[...], 0.0)
    o_ref[...] = t
    if pool_refs:
      p_ref = pool_refs[0]

      @pl.when(pl.program_id(0) == 0)
      def _():
        p_ref[...] = jnp.zeros_like(p_ref)

      p_ref[...] += jnp.sum(t, axis=0, keepdims=True)

  out_shape = [jax.ShapeDtypeStruct((n, h), jnp.float32)]
  out_specs = [pl.BlockSpec((r, h), lambda i: (i, 0))]
  if pool:
    out_shape.append(jax.ShapeDtypeStruct((1, h), jnp.float32))
    out_specs.append(pl.BlockSpec((1, h), lambda i: (0, 0)))

  return pl.pallas_call(
      body,
      grid=(g,),
      in_specs=[
          pl.BlockSpec(memory_space=pltpu.SMEM),
          pl.BlockSpec((r, h), lambda i: (i, 0)),
          pl.BlockSpec((r, h), lambda i: (i, 0)),
          pl.BlockSpec((h, h), lambda i: (0, 0)),
          pl.BlockSpec((1, h), lambda i: (0, 0)),
          pl.BlockSpec((h, h), lambda i: (0, 0)),
          pl.BlockSpec((1, h), lambda i: (0, 0)),
      ],
      out_specs=out_specs,
      out_shape=out_shape,
  )(eps, x, agg, w1, b1, w2, b2)


def _tc_heads(pooled_sum, n, gf, wc, bc, wd1, bd1, wd2, bd2, wd3, bd3,
              wr1, br1, wr2, br2, wr3, br3):
  h = pooled_sum.shape[1]
  d_g = gf.shape[1]
  nd = wd3.shape[1]
  nr = wr3.shape[1]

  def body(ps_ref, gf_ref, wc_ref, bc_ref, wd1_ref, bd1_ref, wd2_ref,
           bd2_ref, wd3_ref, bd3_ref, wr1_ref, br1_ref, wr2_ref, br2_ref,
           wr3_ref, br3_ref, od_ref, or_ref):
    def mm(a, w):
      return jnp.dot(a, w, preferred_element_type=jnp.float32,
                     precision=_PREC)

    pooled = ps_ref[...] * (1.0 / n)
    emb = jnp.maximum(
        mm(pooled, wc_ref[0:h, :]) + mm(gf_ref[...], wc_ref[h:h + d_g, :])
        + bc_ref[...], 0.0)
    d = jnp.maximum(mm(emb, wd1_ref[...]) + bd1_ref[...], 0.0)
    d = jnp.maximum(mm(d, wd2_ref[...]) + bd2_ref[...], 0.0)
    od_ref[...] = mm(d, wd3_ref[...]) + bd3_ref[...]
    rr = jnp.maximum(mm(emb, wr1_ref[...]) + br1_ref[...], 0.0)
    rr = jnp.maximum(mm(rr, wr2_ref[...]) + br2_ref[...], 0.0)
    or_ref[...] = mm(rr, wr3_ref[...]) + br3_ref[...]

  return pl.pallas_call(
      body,
      out_shape=(jax.ShapeDtypeStruct((1, nd), jnp.float32),
                 jax.ShapeDtypeStruct((1, nr), jnp.float32)),
  )(pooled_sum, gf, wc, bc, wd1, bd1, wd2, bd2, wd3, bd3,
    wr1, br1, wr2, br2, wr3, br3)


def kernel(node_features, edge_index, global_features, batch,
           W_in, b_in, gin_eps, gin_W1, gin_b1, gin_W2, gin_b2,
           W_comb, b_comb,
           Wd1, bd1, Wd2, bd2, Wd3, bd3,
           Wr1, br1, Wr2, br2, Wr3, br3):
  n, _ = node_features.shape
  e = edge_index.shape[1]
  h = W_in.shape[1]
  nl = gin_W1.shape[0]

  half = n // 2
  zstride = -(-(half + 1) // (NS * 32)) * 32
  hpad = NS * zstride
  ept0 = -(-e // (NC * NS * 1024)) * 1024
  epad = NC * NS * ept0
  rs = ept0

  src = jnp.concatenate(
      [edge_index[0], jnp.zeros((epad - e,), jnp.int32)])
  dst = jnp.concatenate(
      [edge_index[1], jnp.full((epad - e,), n, jnp.int32)])

  rsrc, rdst, cnts = _sc_route(src, dst, n=n, half=half, ept0=ept0, rs=rs)
  rsrc = rsrc.reshape(-1, S)
  rdst = rdst.reshape(-1, S)

  x = _tc_inproj(node_features, W_in, b_in.reshape(1, h))
  pooled_sum = None
  for l in range(nl):
    agg = _sc_agg_routed(x, rsrc, rdst, cnts, half=half, hpad=hpad,
                         zstride=zstride, rs=rs)
    res = _tc_layer(x, agg, gin_eps[l].reshape(1,),
                    gin_W1[l], gin_b1[l].reshape(1, h),
                    gin_W2[l], gin_b2[l].reshape(1, h),
                    pool=(l == nl - 1))
    if l == nl - 1:
      x, pooled_sum = res
    else:
      (x,) = res

  return _tc_heads(pooled_sum, n, global_features,
                   W_comb, b_comb.reshape(1, h),
                   Wd1, bd1.reshape(1, h), Wd2, bd2.reshape(1, h // 2),
                   Wd3, bd3.reshape(1, Wd3.shape[1]),
                   Wr1, br1.reshape(1, h), Wr2, br2.reshape(1, h // 2),
                   Wr3, br3.reshape(1, Wr3.shape[1]))
